# baseline (speedup 1.0000x reference)
import functools

import jax
import jax.numpy as jnp
from jax import lax
from jax.experimental import pallas as pl
from jax.experimental.pallas import tpu as pltpu
from jax.experimental.pallas import tpu_sc as plsc

N = 10000
E = 320000
D = 128
C = 64
NP = 10240
RPT = NP // 16
NC = 2
NS = 16
NW = NC * NS
EPW = E // NW
CH = 125
NCH = EPW // CH
BM = 2048

_sc_mesh = plsc.VectorSubcoreMesh(core_axis_name="c", subcore_axis_name="s")


@functools.partial(
    pl.kernel,
    out_type=(jax.ShapeDtypeStruct((NP,), jnp.float32),
              jax.ShapeDtypeStruct((NP,), jnp.float32)),
    mesh=_sc_mesh,
    scratch_types=[
        pltpu.VMEM((NCH, CH), jnp.int32),
        pltpu.VMEM((CH,), jnp.float32),
        pltpu.VMEM_SHARED((NP,), jnp.float32),
        [pltpu.SemaphoreType.DMA for _ in range(8)],
    ],
    compiler_params=pltpu.CompilerParams(use_tc_tiling_on_sc=False),
)
def _deg_call(ei_hbm, ones_hbm, zer_hbm, out0, out1, idx_c, ones_v, acc, ssem):
    NB = 8
    c = lax.axis_index("c")
    s = lax.axis_index("s")
    wid = c * NS + s
    pltpu.sync_copy(zer_hbm.at[pl.ds(s * RPT, RPT)], acc.at[pl.ds(s * RPT, RPT)])
    pltpu.sync_copy(ei_hbm.at[1, wid], idx_c)
    pltpu.sync_copy(ones_hbm, ones_v)
    plsc.subcore_barrier()

    for k in range(NB):
        pltpu.async_copy(ones_v, acc.at[idx_c.at[k]], ssem[k], add=True)

    def round_(i, carry):
        for k in range(NB):
            j = i * NB + k
            pltpu.make_async_copy(ones_v, acc.at[idx_c.at[j]], ssem[k]).wait()
            pltpu.async_copy(ones_v, acc.at[idx_c.at[j + NB]], ssem[k], add=True)
        return carry

    lax.fori_loop(0, NCH // NB - 1, round_, 0)
    for k in range(NB):
        pltpu.make_async_copy(ones_v, acc.at[idx_c.at[k]], ssem[k]).wait()
    plsc.subcore_barrier()

    @pl.when(c == 0)
    def _():
        pltpu.sync_copy(acc.at[pl.ds(s * RPT, RPT)], out0.at[pl.ds(s * RPT, RPT)])

    @pl.when(c == 1)
    def _():
        pltpu.sync_copy(acc.at[pl.ds(s * RPT, RPT)], out1.at[pl.ds(s * RPT, RPT)])


@functools.partial(
    pl.kernel,
    out_type=(jax.ShapeDtypeStruct((NP, C), jnp.float32),
              jax.ShapeDtypeStruct((NP, C), jnp.float32)),
    mesh=_sc_mesh,
    scratch_types=[
        pltpu.VMEM((NCH, CH), jnp.int32),
        pltpu.VMEM((NCH, CH), jnp.int32),
        [pltpu.VMEM((CH, C), jnp.float32) for _ in range(8)],
        pltpu.VMEM_SHARED((NP, C), jnp.float32),
        [pltpu.SemaphoreType.DMA for _ in range(8)],
        [pltpu.SemaphoreType.DMA for _ in range(8)],
    ],
    compiler_params=pltpu.CompilerParams(use_tc_tiling_on_sc=False),
)
def _hop_call(g_hbm, ei_hbm, zer_hbm, out0, out1,
              idx_r, idx_c, bufs, acc, gsem, ssem):
    NB = 8
    c = lax.axis_index("c")
    s = lax.axis_index("s")
    wid = c * NS + s
    pltpu.sync_copy(zer_hbm, acc.at[pl.ds(s * RPT, RPT)])
    pltpu.sync_copy(ei_hbm.at[0, wid], idx_r)
    pltpu.sync_copy(ei_hbm.at[1, wid], idx_c)
    plsc.subcore_barrier()

    for k in range(NB):
        pltpu.async_copy(g_hbm.at[idx_r.at[k]], bufs[k], gsem[k])

    def round_(i, carry):
        for k in range(NB):
            j = i * NB + k
            pltpu.make_async_copy(g_hbm.at[idx_r.at[j]], bufs[k], gsem[k]).wait()
            pltpu.async_copy(bufs[k], acc.at[idx_c.at[j]], ssem[k], add=True)
        for k in range(NB):
            j = i * NB + k
            jn = lax.min(j + NB, NCH - 1)
            pltpu.make_async_copy(bufs[k], acc.at[idx_c.at[j]], ssem[k]).wait()
            pltpu.async_copy(g_hbm.at[idx_r.at[jn]], bufs[k], gsem[k])
        return carry

    lax.fori_loop(0, NCH // NB, round_, 0)
    for k in range(NB):
        pltpu.make_async_copy(g_hbm.at[idx_r.at[0]], bufs[k], gsem[k]).wait()
    plsc.subcore_barrier()

    @pl.when(c == 0)
    def _():
        pltpu.sync_copy(acc.at[pl.ds(s * RPT, RPT)], out0.at[pl.ds(s * RPT, RPT)])

    @pl.when(c == 1)
    def _():
        pltpu.sync_copy(acc.at[pl.ds(s * RPT, RPT)], out1.at[pl.ds(s * RPT, RPT)])


NPW = NP // 2
WC = 2 * C


def _dis128(dis2):
    l = jnp.broadcast_to(dis2[:, 0:1], (dis2.shape[0], C))
    r = jnp.broadcast_to(dis2[:, 1:2], (dis2.shape[0], C))
    return jnp.concatenate([l, r], axis=1)


def _lin_body(x_ref, w2_ref, d0_ref, d1_ref, g_ref, dis_ref):
    deg = d0_ref[...] + d1_ref[...] + 1.0
    dis = lax.rsqrt(deg)
    z = lax.dot_general(x_ref[...], w2_ref[...], (((1,), (0,)), ((), ())),
                        preferred_element_type=jnp.float32)
    g_ref[...] = z * _dis128(dis)
    dis_ref[...] = dis


def _lin_call(xw, W2, d0, d1):
    BLW = 1000
    return pl.pallas_call(
        _lin_body,
        grid=(N // (2 * BLW),),
        in_specs=[pl.BlockSpec((BLW, 2 * D), lambda i: (i, 0)),
                  pl.BlockSpec((2 * D, WC), lambda i: (0, 0)),
                  pl.BlockSpec((BLW, 2), lambda i: (i, 0)),
                  pl.BlockSpec((BLW, 2), lambda i: (i, 0))],
        out_specs=[pl.BlockSpec((BLW, WC), lambda i: (i, 0)),
                   pl.BlockSpec((BLW, 2), lambda i: (i, 0))],
        out_shape=[jax.ShapeDtypeStruct((NPW, WC), jnp.float32),
                   jax.ShapeDtypeStruct((NPW, 2), jnp.float32)],
    )(xw, W2, d0, d1)


def _comb_body(p0_ref, p1_ref, g_ref, dis_ref, out_ref):
    d = _dis128(dis_ref[...])
    out_ref[...] = (d * d) * (p0_ref[...] + p1_ref[...] + g_ref[...])


def _comb_call(p0, p1, g0, dis):
    BMW = BM // 2
    return pl.pallas_call(
        _comb_body,
        grid=(NPW // BMW,),
        in_specs=[pl.BlockSpec((BMW, WC), lambda i: (i, 0)),
                  pl.BlockSpec((BMW, WC), lambda i: (i, 0)),
                  pl.BlockSpec((BMW, WC), lambda i: (i, 0)),
                  pl.BlockSpec((BMW, 2), lambda i: (i, 0))],
        out_specs=pl.BlockSpec((BMW, WC), lambda i: (i, 0)),
        out_shape=jax.ShapeDtypeStruct((NPW, WC), jnp.float32),
    )(p0, p1, g0, dis)


def _final_body(p0_ref, p1_ref, g_ref, dis_ref, b_ref, out_ref):
    y = _dis128(dis_ref[...]) * (p0_ref[...] + p1_ref[...] + g_ref[...]) + b_ref[...]
    yl = y[:, :C]
    yr = y[:, C:]

    def lsm(h):
        m = jnp.max(h, axis=1, keepdims=True)
        e = jnp.exp(h - m)
        return (h - m) - jnp.log(jnp.sum(e, axis=1, keepdims=True))

    out_ref[...] = jnp.concatenate([lsm(yl), lsm(yr)], axis=1)


def _final_call(p0, p1, g1, dis, b2):
    BFW = 1000
    return pl.pallas_call(
        _final_body,
        grid=(N // (2 * BFW),),
        in_specs=[pl.BlockSpec((BFW, WC), lambda i: (i, 0)),
                  pl.BlockSpec((BFW, WC), lambda i: (i, 0)),
                  pl.BlockSpec((BFW, WC), lambda i: (i, 0)),
                  pl.BlockSpec((BFW, 2), lambda i: (i, 0)),
                  pl.BlockSpec((1, WC), lambda i: (0, 0))],
        out_specs=pl.BlockSpec((BFW, WC), lambda i: (i, 0)),
        out_shape=jax.ShapeDtypeStruct((N // 2, WC), jnp.float32),
    )(p0, p1, g1, dis, b2)


def kernel(x, edge_index, W, b):
    e4 = edge_index.reshape(2, NW, NCH, CH)
    xw = x.reshape(N // 2, 2 * D)
    W2 = jnp.zeros((2 * D, WC), jnp.float32)
    W2 = W2.at[:D, :C].set(W.T).at[D:, C:].set(W.T)
    ones_ch = jnp.ones((CH,), jnp.float32)
    zer1 = jnp.zeros((NP,), jnp.float32)

    dp0, dp1 = _deg_call(e4, ones_ch, zer1)
    g0w, dis = _lin_call(xw, W2, dp0.reshape(NPW, 2), dp1.reshape(NPW, 2))
    zer_t = jnp.zeros((RPT, C), jnp.float32)
    p0, p1 = _hop_call(g0w.reshape(NP, C), e4, zer_t)
    g1w = _comb_call(p0.reshape(NPW, WC), p1.reshape(NPW, WC), g0w, dis)
    q0, q1 = _hop_call(g1w.reshape(NP, C), e4, zer_t)
    outw = _final_call(q0.reshape(NPW, WC), q1.reshape(NPW, WC),
                       g1w, dis, jnp.concatenate([b, b]).reshape(1, WC))
    return outw.reshape(N, C)


# speedup vs baseline: 1.0068x; 1.0068x over previous
import functools

import jax
import jax.numpy as jnp
from jax import lax
from jax.experimental import pallas as pl
from jax.experimental.pallas import tpu as pltpu
from jax.experimental.pallas import tpu_sc as plsc

N = 10000
E = 320000
D = 128
C = 64
NP = 10240
RPT = NP // 16
NC = 2
NS = 16
NW = NC * NS
EPW = E // NW
CH = 125
NCH = EPW // CH
BM = 2048

_sc_mesh = plsc.VectorSubcoreMesh(core_axis_name="c", subcore_axis_name="s")


@functools.partial(
    pl.kernel,
    out_type=(jax.ShapeDtypeStruct((NP,), jnp.float32),
              jax.ShapeDtypeStruct((NP,), jnp.float32)),
    mesh=_sc_mesh,
    scratch_types=[
        pltpu.VMEM((NCH, CH), jnp.int32),
        pltpu.VMEM((CH,), jnp.float32),
        pltpu.VMEM_SHARED((NP,), jnp.float32),
        [pltpu.SemaphoreType.DMA for _ in range(8)],
    ],
    compiler_params=pltpu.CompilerParams(use_tc_tiling_on_sc=False),
)
def _deg_call(ei_hbm, ones_hbm, zer_hbm, out0, out1, idx_c, ones_v, acc, ssem):
    NB = 8
    c = lax.axis_index("c")
    s = lax.axis_index("s")
    wid = c * NS + s
    pltpu.sync_copy(zer_hbm.at[pl.ds(s * RPT, RPT)], acc.at[pl.ds(s * RPT, RPT)])
    pltpu.sync_copy(ei_hbm.at[1, wid], idx_c)
    pltpu.sync_copy(ones_hbm, ones_v)
    plsc.subcore_barrier()

    for k in range(NB):
        pltpu.async_copy(ones_v, acc.at[idx_c.at[k]], ssem[k], add=True)

    def round_(i, carry):
        for k in range(NB):
            j = i * NB + k
            pltpu.make_async_copy(ones_v, acc.at[idx_c.at[j]], ssem[k]).wait()
            pltpu.async_copy(ones_v, acc.at[idx_c.at[j + NB]], ssem[k], add=True)
        return carry

    lax.fori_loop(0, NCH // NB - 1, round_, 0)
    for k in range(NB):
        pltpu.make_async_copy(ones_v, acc.at[idx_c.at[k]], ssem[k]).wait()
    plsc.subcore_barrier()

    @pl.when(c == 0)
    def _():
        pltpu.sync_copy(acc.at[pl.ds(s * RPT, RPT)], out0.at[pl.ds(s * RPT, RPT)])

    @pl.when(c == 1)
    def _():
        pltpu.sync_copy(acc.at[pl.ds(s * RPT, RPT)], out1.at[pl.ds(s * RPT, RPT)])


@functools.partial(
    pl.kernel,
    out_type=(jax.ShapeDtypeStruct((NP, C), jnp.float32),
              jax.ShapeDtypeStruct((NP, C), jnp.float32)),
    mesh=_sc_mesh,
    scratch_types=[
        pltpu.VMEM((NCH, CH), jnp.int32),
        pltpu.VMEM((NCH, CH), jnp.int32),
        [pltpu.VMEM((CH, C), jnp.float32) for _ in range(8)],
        pltpu.VMEM_SHARED((NP, C), jnp.float32),
        [pltpu.SemaphoreType.DMA for _ in range(8)],
        [pltpu.SemaphoreType.DMA for _ in range(8)],
    ],
    compiler_params=pltpu.CompilerParams(use_tc_tiling_on_sc=False),
)
def _hop_call(g_hbm, ei_hbm, zer_hbm, out0, out1,
              idx_r, idx_c, bufs, acc, gsem, ssem):
    NB = 8
    c = lax.axis_index("c")
    s = lax.axis_index("s")
    wid = c * NS + s
    pltpu.sync_copy(zer_hbm, acc.at[pl.ds(s * RPT, RPT)])
    pltpu.sync_copy(ei_hbm.at[0, wid], idx_r)
    pltpu.sync_copy(ei_hbm.at[1, wid], idx_c)
    plsc.subcore_barrier()

    for k in range(NB):
        pltpu.async_copy(g_hbm.at[idx_r.at[k]], bufs[k], gsem[k])

    def round_(i, carry):
        for k in range(NB):
            j = i * NB + k
            pltpu.make_async_copy(g_hbm.at[idx_r.at[j]], bufs[k], gsem[k]).wait()
            pltpu.async_copy(bufs[k], acc.at[idx_c.at[j]], ssem[k], add=True)
        for k in range(NB):
            j = i * NB + k
            jn = lax.min(j + NB, NCH - 1)
            pltpu.make_async_copy(bufs[k], acc.at[idx_c.at[j]], ssem[k]).wait()
            pltpu.async_copy(g_hbm.at[idx_r.at[jn]], bufs[k], gsem[k])
        return carry

    lax.fori_loop(0, NCH // NB, round_, 0)
    for k in range(NB):
        pltpu.make_async_copy(g_hbm.at[idx_r.at[0]], bufs[k], gsem[k]).wait()
    plsc.subcore_barrier()

    @pl.when(c == 0)
    def _():
        pltpu.sync_copy(acc.at[pl.ds(s * RPT, RPT)], out0.at[pl.ds(s * RPT, RPT)])

    @pl.when(c == 1)
    def _():
        pltpu.sync_copy(acc.at[pl.ds(s * RPT, RPT)], out1.at[pl.ds(s * RPT, RPT)])


NPW = NP // 2
WC = 2 * C


def _dis128(dis2):
    l = jnp.broadcast_to(dis2[:, 0:1], (dis2.shape[0], C))
    r = jnp.broadcast_to(dis2[:, 1:2], (dis2.shape[0], C))
    return jnp.concatenate([l, r], axis=1)


def _lin_body(x_ref, w2_ref, d0_ref, d1_ref, g_ref, dis_ref):
    deg = d0_ref[...] + d1_ref[...] + 1.0
    dis = lax.rsqrt(deg)
    z = lax.dot_general(x_ref[...], w2_ref[...], (((1,), (0,)), ((), ())),
                        preferred_element_type=jnp.float32)
    g_ref[...] = z * _dis128(dis)
    dis_ref[...] = dis


def _lin_call(xw, W2, d0, d1):
    BLW = BM // 2
    return pl.pallas_call(
        _lin_body,
        grid=(NPW // BLW,),
        in_specs=[pl.BlockSpec((BLW, 2 * D), lambda i: (i, 0)),
                  pl.BlockSpec((2 * D, WC), lambda i: (0, 0)),
                  pl.BlockSpec((BLW, 2), lambda i: (i, 0)),
                  pl.BlockSpec((BLW, 2), lambda i: (i, 0))],
        out_specs=[pl.BlockSpec((BLW, WC), lambda i: (i, 0)),
                   pl.BlockSpec((BLW, 2), lambda i: (i, 0))],
        out_shape=[jax.ShapeDtypeStruct((NPW, WC), jnp.float32),
                   jax.ShapeDtypeStruct((NPW, 2), jnp.float32)],
    )(xw, W2, d0, d1)


def _comb_body(p0_ref, p1_ref, g_ref, dis_ref, out_ref):
    d = _dis128(dis_ref[...])
    out_ref[...] = (d * d) * (p0_ref[...] + p1_ref[...] + g_ref[...])


def _comb_call(p0, p1, g0, dis):
    BMW = BM // 2
    return pl.pallas_call(
        _comb_body,
        grid=(NPW // BMW,),
        in_specs=[pl.BlockSpec((BMW, WC), lambda i: (i, 0)),
                  pl.BlockSpec((BMW, WC), lambda i: (i, 0)),
                  pl.BlockSpec((BMW, WC), lambda i: (i, 0)),
                  pl.BlockSpec((BMW, 2), lambda i: (i, 0))],
        out_specs=pl.BlockSpec((BMW, WC), lambda i: (i, 0)),
        out_shape=jax.ShapeDtypeStruct((NPW, WC), jnp.float32),
    )(p0, p1, g0, dis)


def _final_body(p0_ref, p1_ref, g_ref, dis_ref, b_ref, out_ref):
    y = _dis128(dis_ref[...]) * (p0_ref[...] + p1_ref[...] + g_ref[...]) + b_ref[...]
    yl = y[:, :C]
    yr = y[:, C:]

    def lsm(h):
        m = jnp.max(h, axis=1, keepdims=True)
        e = jnp.exp(h - m)
        return (h - m) - jnp.log(jnp.sum(e, axis=1, keepdims=True))

    out_ref[...] = jnp.concatenate([lsm(yl), lsm(yr)], axis=1)


def _final_call(p0, p1, g1, dis, b2):
    BFW = 1000
    return pl.pallas_call(
        _final_body,
        grid=(N // (2 * BFW),),
        in_specs=[pl.BlockSpec((BFW, WC), lambda i: (i, 0)),
                  pl.BlockSpec((BFW, WC), lambda i: (i, 0)),
                  pl.BlockSpec((BFW, WC), lambda i: (i, 0)),
                  pl.BlockSpec((BFW, 2), lambda i: (i, 0)),
                  pl.BlockSpec((1, WC), lambda i: (0, 0))],
        out_specs=pl.BlockSpec((BFW, WC), lambda i: (i, 0)),
        out_shape=jax.ShapeDtypeStruct((N // 2, WC), jnp.float32),
    )(p0, p1, g1, dis, b2)


def kernel(x, edge_index, W, b):
    e4 = edge_index.reshape(2, NW, NCH, CH)
    xw = jnp.pad(x, ((0, NP - N), (0, 0))).reshape(NPW, 2 * D)
    W2 = jnp.zeros((2 * D, WC), jnp.float32)
    W2 = W2.at[:D, :C].set(W.T).at[D:, C:].set(W.T)
    ones_ch = jnp.ones((CH,), jnp.float32)
    zer1 = jnp.zeros((NP,), jnp.float32)

    dp0, dp1 = _deg_call(e4, ones_ch, zer1)
    g0w, dis = _lin_call(xw, W2, dp0.reshape(NPW, 2), dp1.reshape(NPW, 2))
    zer_t = jnp.zeros((RPT, C), jnp.float32)
    p0, p1 = _hop_call(g0w.reshape(NP, C), e4, zer_t)
    g1w = _comb_call(p0.reshape(NPW, WC), p1.reshape(NPW, WC), g0w, dis)
    q0, q1 = _hop_call(g1w.reshape(NP, C), e4, zer_t)
    outw = _final_call(q0.reshape(NPW, WC), q1.reshape(NPW, WC),
                       g1w, dis, jnp.concatenate([b, b]).reshape(1, WC))
    return outw.reshape(N, C)
